# baseline (speedup 1.0000x reference)
import functools

import jax
import jax.numpy as jnp
from jax import lax
from jax.experimental import pallas as pl
from jax.experimental.pallas import tpu as pltpu
from jax.experimental.pallas import tpu_sc as plsc

NC = 2
NS = 16
NW = NC * NS

TBLK = 65536


def _transpose_body(x_ref, o_ref):
    for k in range(TBLK // 512):
        x3 = x_ref[:, 512 * k:512 * (k + 1)].reshape(32, 4, 128)
        a = x3.transpose(1, 0, 2).reshape(128, 128)
        o_ref[pl.ds(128 * k, 128), :] = a.T


@functools.lru_cache(maxsize=None)
def _build_transpose(V: int, D: int):
    grid = (V + TBLK - 1) // TBLK
    rows_blk = TBLK // 4
    return pl.pallas_call(
        _transpose_body,
        grid=(grid,),
        in_specs=[pl.BlockSpec((D, TBLK), lambda g: (0, g))],
        out_specs=pl.BlockSpec((rows_blk, 128), lambda g: (g, 0)),
        out_shape=jax.ShapeDtypeStruct((grid * rows_blk, 128),
                                       jnp.float32),
    )


def _outpack_body(x_ref, o_ref):
    FB, Q, _ = x_ref.shape
    D = o_ref.shape[1]
    for ff in range(FB):
        for j in range(Q // 128):
            zt = x_ref[ff, pl.ds(128 * j, 128), :].T
            for u in range(128 // D):
                o_ref[ff, :, pl.ds(Q * u + 128 * j, 128)] = (
                    zt[D * u:D * (u + 1), :])


@functools.lru_cache(maxsize=None)
def _build_outpack(B: int, F: int, D: int):
    Q = B * D // 128
    FB = 2
    assert F % FB == 0
    return pl.pallas_call(
        _outpack_body,
        grid=(F // FB,),
        in_specs=[pl.BlockSpec((FB, Q, 128), lambda f: (f, 0, 0))],
        out_specs=pl.BlockSpec((FB, D, B), lambda f: (f, 0, 0)),
        out_shape=jax.ShapeDtypeStruct((F, D, B), jnp.float32),
    )


@functools.lru_cache(maxsize=None)
def _build_gather(B: int, F: int, VP: int, D: int):
    assert B % NW == 0 and F % 2 == 0
    C = B // NW
    Q = B * D // 128
    mesh = plsc.VectorSubcoreMesh(core_axis_name="c", subcore_axis_name="s")

    @functools.partial(
        pl.kernel,
        mesh=mesh,
        out_type=jax.ShapeDtypeStruct((F, Q, 128), jnp.float32),
        compiler_params=pltpu.CompilerParams(use_tc_tiling_on_sc=False,
                                             needs_layout_passes=False),
        scratch_types=[
            pltpu.VMEM((C,), jnp.int32),
            pltpu.VMEM((C,), jnp.int32),
            pltpu.VMEM((C, D), jnp.float32),
            pltpu.VMEM((C, D), jnp.float32),
            pltpu.SemaphoreType.DMA,
            pltpu.SemaphoreType.DMA,
            pltpu.SemaphoreType.DMA,
            pltpu.SemaphoreType.DMA,
        ],
    )
    def gather_kernel(idx_hbm, table_hbm, out_hbm, idx_a, idx_b,
                      rows_a, rows_b, gsem0, gsem1, ssem0, ssem1):
        wid = lax.axis_index("s") * NC + lax.axis_index("c")
        b0 = wid * C
        u = wid // (Q // C)
        q0 = (wid % (Q // C)) * C
        idx_v = [idx_a, idx_b]
        rows_v = [rows_a, rows_b]
        gsem = [gsem0, gsem1]
        ssem = [ssem0, ssem1]

        def permute(ref):
            def body(i, _):
                v = ref[pl.ds(i * 16, 16)]
                m = (v & -512) | ((v & 127) << 2) | ((v >> 7) & 3)
                ref[pl.ds(i * 16, 16)] = m
                return _
            lax.fori_loop(0, C // 16, body, None)

        def issue_gather(f_next, nbuf):
            pltpu.sync_copy(idx_hbm.at[f_next, pl.ds(b0, C)], idx_v[nbuf])
            permute(idx_v[nbuf])
            pltpu.async_copy(table_hbm.at[idx_v[nbuf]], rows_v[nbuf],
                             gsem[nbuf])

        def wait_gather(buf):
            pltpu.make_async_copy(table_hbm.at[idx_v[buf]], rows_v[buf],
                                  gsem[buf]).wait()

        def store(f, buf):
            pltpu.async_copy(
                rows_v[buf],
                out_hbm.at[f, pl.ds(q0, C), pl.ds(u * D, D)], ssem[buf])

        def wait_store(f_old, buf):
            pltpu.make_async_copy(
                rows_v[buf],
                out_hbm.at[f_old, pl.ds(q0, C), pl.ds(u * D, D)],
                ssem[buf]).wait()

        issue_gather(0, 0)

        def pair_body(p, carry):
            for b in range(2):
                f = 2 * p + b
                if b == 0:
                    @pl.when(p > 0)
                    def _():
                        wait_store(f - 1, 1)
                else:
                    wait_store(f - 1, 0)
                issue_gather(f + 1, 1 - b)
                wait_gather(b)
                store(f, b)
            return carry

        lax.fori_loop(0, F // 2 - 1, pair_body, None)
        for f in (F - 2, F - 1):
            b = f % 2
            wait_store(f - 1, 1 - b)
            if f + 1 < F:
                issue_gather(f + 1, 1 - b)
            wait_gather(b)
            store(f, b)
        wait_store(F - 1, 1)

    return gather_kernel


def kernel(global_indices, table):
    B, F = global_indices.shape
    V, D = table.shape
    tt = table.T
    scratch = _build_transpose(V, D)(tt)
    tlin = scratch.reshape(-1, D)
    idxT = global_indices.T.astype(jnp.int32)
    pack = _build_gather(B, F, tlin.shape[0], D)(idxT, tlin)
    out3 = _build_outpack(B, F, D)(pack)
    return jnp.transpose(out3, (2, 0, 1))
